# baseline (speedup 1.0000x reference)
import functools

import jax
import jax.numpy as jnp
from jax import lax
from jax.experimental import pallas as pl
from jax.experimental.pallas import tpu as pltpu
from jax.experimental.pallas import tpu_sc as plsc

FIELD = 26
D = 64
PAIRS = FIELD * (FIELD - 1) // 2
ROW = FIELD * D
B = 4096
NC, NS = 2, 16
NW = NC * NS
BPW = B // NW
CHUNK = 16
NCHUNK = BPW // CHUNK

_PAIRS_IJ = [(i, j) for i in range(FIELD) for j in range(i + 1, FIELD)]


def _sc_body(x_hbm, k_hbm, o_hbm, x_v0, x_v1, k_v, o_v, t_v, sx0, sx1, so):
    wid = lax.axis_index("s") * NC + lax.axis_index("c")
    base = wid * BPW
    pltpu.sync_copy(k_hbm, k_v)
    iota = lax.iota(jnp.int32, 16)
    row_base = iota * ROW
    tp_base = iota * CHUNK
    zvec = jnp.zeros((16,), jnp.float32)

    def x_copy(c, b):
        sem = sx0 if b == 0 else sx1
        xv = x_v0 if b == 0 else x_v1
        return pltpu.make_async_copy(
            x_hbm.at[pl.ds((base + c * CHUNK) * ROW, CHUNK * ROW)],
            xv,
            sem)

    x_copy(0, 0).start()
    x_copy(1, 1).start()

    def run_chunk(c, b):
        x_copy(c, b).wait()

        def zbody(q, _):
            o_v[pl.ds(q * 16, 16)] = zvec
            return 0

        lax.fori_loop(0, PAIRS, zbody, 0)

        def dbody(d, _):
            xv = x_v0 if b == 0 else x_v1
            vx = [plsc.load_gather(xv, [row_base + (f * D) + d])
                  for f in range(FIELD)]
            dsplat = jnp.zeros((16,), jnp.int32) + d
            for p, (i, j) in enumerate(_PAIRS_IJ):
                kv = plsc.load_gather(k_v, [dsplat + (p * D)])
                m = vx[i] * vx[j] * kv
                plsc.addupdate(o_v.at[pl.ds(p * 16, 16)], m)
            return 0

        lax.fori_loop(0, D, dbody, 0)

        for bl in range(CHUNK):
            for pc in range(0, PAIRS, 16):
                n = min(16, PAIRS - pc)
                idxv = tp_base + (pc * CHUNK + bl)
                if n == 16:
                    t_v[pl.ds(bl * PAIRS + pc, 16)] = plsc.load_gather(
                        o_v, [idxv])
                else:
                    mask = iota < n
                    v = plsc.load_gather(o_v, [idxv], mask=mask)
                    plsc.store_compressed(
                        t_v.at[pl.ds(bl * PAIRS + pc, 16)], v, mask=mask)

        cp = pltpu.make_async_copy(
            t_v.at[pl.ds(0, CHUNK * PAIRS)],
            o_hbm.at[pl.ds((base + c * CHUNK) * PAIRS, CHUNK * PAIRS)],
            so)
        cp.start()
        cp.wait()

        nxt = c + 2

        @pl.when(nxt < NCHUNK)
        def _():
            x_copy(nxt, b).start()

    def outer(cc, _):
        run_chunk(cc * 2, 0)
        run_chunk(cc * 2 + 1, 1)
        return 0

    lax.fori_loop(0, NCHUNK // 2, outer, 0)


_sc_call = pl.kernel(
    _sc_body,
    out_type=jax.ShapeDtypeStruct((B * PAIRS,), jnp.float32),
    mesh=plsc.VectorSubcoreMesh(core_axis_name="c", subcore_axis_name="s"),
    compiler_params=pltpu.CompilerParams(needs_layout_passes=False),
    scratch_types=[
        pltpu.VMEM((CHUNK * ROW,), jnp.float32),
        pltpu.VMEM((CHUNK * ROW,), jnp.float32),
        pltpu.VMEM((PAIRS * D,), jnp.float32),
        pltpu.VMEM((PAIRS * CHUNK,), jnp.float32),
        pltpu.VMEM((CHUNK * PAIRS + 16,), jnp.float32),
        pltpu.SemaphoreType.DMA,
        pltpu.SemaphoreType.DMA,
        pltpu.SemaphoreType.DMA,
    ],
)


def kernel(inputs, kernel, training=False):
    b = inputs.shape[0]
    assert b == B and inputs.shape[1] == FIELD and inputs.shape[2] == D
    out = _sc_call(inputs.reshape(-1), kernel.reshape(-1))
    return out.reshape(b, PAIRS)


# speedup vs baseline: 3.5172x; 3.5172x over previous
import functools

import jax
import jax.numpy as jnp
from jax import lax
from jax.experimental import pallas as pl
from jax.experimental.pallas import tpu as pltpu
from jax.experimental.pallas import tpu_sc as plsc

FIELD = 26
D = 64
PAIRS = FIELD * (FIELD - 1) // 2
ROW = FIELD * D
B = 4096
NC, NS = 2, 16
NW = NC * NS
BPW = B // NW
CHUNK = 16
NCHUNK = BPW // CHUNK

_PAIRS_IJ = [(i, j) for i in range(FIELD) for j in range(i + 1, FIELD)]


def _sc_body(x_hbm, k_hbm, o_hbm, x_v0, x_v1, k_v, o_v, t_v, sx0, sx1, so):
    wid = lax.axis_index("s") * NC + lax.axis_index("c")
    base = wid * BPW
    pltpu.sync_copy(k_hbm, k_v)
    iota = lax.iota(jnp.int32, 16)
    row_base = iota * ROW
    tp_base = iota * CHUNK
    zvec = jnp.zeros((16,), jnp.float32)

    def x_copy(c, b):
        sem = sx0 if b == 0 else sx1
        xv = x_v0 if b == 0 else x_v1
        return pltpu.make_async_copy(
            x_hbm.at[pl.ds((base + c * CHUNK) * ROW, CHUNK * ROW)],
            xv,
            sem)

    x_copy(0, 0).start()
    x_copy(1, 1).start()

    def run_chunk(c, b):
        x_copy(c, b).wait()

        def zbody(q, _):
            o_v[pl.ds(q * 16, 16)] = zvec
            return 0

        lax.fori_loop(0, PAIRS, zbody, 0)

        def dbody(d, _):
            xv = x_v0 if b == 0 else x_v1
            vx = [plsc.load_gather(xv, [row_base + (f * D) + d])
                  for f in range(FIELD)]
            dsplat = jnp.zeros((16,), jnp.int32) + d
            for p0 in range(0, PAIRS, 8):
                blk = range(p0, min(p0 + 8, PAIRS))
                kvs = [plsc.load_gather(k_v, [dsplat + (p * D)])
                       for p in blk]
                ts = [vx[_PAIRS_IJ[p][0]] * vx[_PAIRS_IJ[p][1]] for p in blk]
                ms = [t * kv for t, kv in zip(ts, kvs)]
                for p, m in zip(blk, ms):
                    plsc.addupdate(o_v.at[pl.ds(p * 16, 16)], m)
            return 0

        lax.fori_loop(0, D, dbody, 0)

        for bl in range(CHUNK):
            for pc in range(0, PAIRS, 16):
                n = min(16, PAIRS - pc)
                idxv = tp_base + (pc * CHUNK + bl)
                if n == 16:
                    t_v[pl.ds(bl * PAIRS + pc, 16)] = plsc.load_gather(
                        o_v, [idxv])
                else:
                    mask = iota < n
                    v = plsc.load_gather(o_v, [idxv], mask=mask)
                    plsc.store_compressed(
                        t_v.at[pl.ds(bl * PAIRS + pc, 16)], v, mask=mask)

        cp = pltpu.make_async_copy(
            t_v.at[pl.ds(0, CHUNK * PAIRS)],
            o_hbm.at[pl.ds((base + c * CHUNK) * PAIRS, CHUNK * PAIRS)],
            so)
        cp.start()
        cp.wait()

        nxt = c + 2

        @pl.when(nxt < NCHUNK)
        def _():
            x_copy(nxt, b).start()

    def outer(cc, _):
        run_chunk(cc * 2, 0)
        run_chunk(cc * 2 + 1, 1)
        return 0

    lax.fori_loop(0, NCHUNK // 2, outer, 0)


_sc_call = pl.kernel(
    _sc_body,
    out_type=jax.ShapeDtypeStruct((B * PAIRS,), jnp.float32),
    mesh=plsc.VectorSubcoreMesh(core_axis_name="c", subcore_axis_name="s"),
    compiler_params=pltpu.CompilerParams(needs_layout_passes=False),
    scratch_types=[
        pltpu.VMEM((CHUNK * ROW,), jnp.float32),
        pltpu.VMEM((CHUNK * ROW,), jnp.float32),
        pltpu.VMEM((PAIRS * D,), jnp.float32),
        pltpu.VMEM((PAIRS * CHUNK,), jnp.float32),
        pltpu.VMEM((CHUNK * PAIRS + 16,), jnp.float32),
        pltpu.SemaphoreType.DMA,
        pltpu.SemaphoreType.DMA,
        pltpu.SemaphoreType.DMA,
    ],
)


def kernel(inputs, kernel, training=False):
    b = inputs.shape[0]
    assert b == B and inputs.shape[1] == FIELD and inputs.shape[2] == D
    out = _sc_call(inputs.reshape(-1), kernel.reshape(-1))
    return out.reshape(b, PAIRS)


# speedup vs baseline: 3.8331x; 1.0898x over previous
import functools

import jax
import jax.numpy as jnp
from jax import lax
from jax.experimental import pallas as pl
from jax.experimental.pallas import tpu as pltpu
from jax.experimental.pallas import tpu_sc as plsc

FIELD = 26
D = 64
PAIRS = FIELD * (FIELD - 1) // 2
ROW = FIELD * D
B = 4096
NC, NS = 2, 16
NW = NC * NS
BPW = B // NW
CHUNK = 16
NCHUNK = BPW // CHUNK

_PAIRS_IJ = [(i, j) for i in range(FIELD) for j in range(i + 1, FIELD)]


def _sc_body(x_hbm, k_hbm, o_hbm, x_v0, x_v1, k_v, o_v, t_v, sx0, sx1, so):
    wid = lax.axis_index("s") * NC + lax.axis_index("c")
    base = wid * BPW
    pltpu.sync_copy(k_hbm, k_v)
    iota = lax.iota(jnp.int32, 16)
    row_base = iota * ROW
    tp_base = iota * CHUNK
    zvec = jnp.zeros((16,), jnp.float32)

    def x_copy(c, b):
        sem = sx0 if b == 0 else sx1
        xv = x_v0 if b == 0 else x_v1
        return pltpu.make_async_copy(
            x_hbm.at[pl.ds((base + c * CHUNK) * ROW, CHUNK * ROW)],
            xv,
            sem)

    x_copy(0, 0).start()
    x_copy(1, 1).start()

    def run_chunk(c, b):
        x_copy(c, b).wait()

        def zbody(q, _):
            o_v[pl.ds(q * 16, 16)] = zvec
            return 0

        lax.fori_loop(0, PAIRS, zbody, 0)

        @plsc.parallel_loop(0, D, unroll=2)
        def dbody(d):
            xv = x_v0 if b == 0 else x_v1
            vx = [plsc.load_gather(xv, [row_base + (f * D) + d])
                  for f in range(FIELD)]
            dP = d * PAIRS
            for p0 in range(0, PAIRS, 16):
                blk = range(p0, min(p0 + 16, PAIRS))
                kblk = k_v[pl.ds(dP + p0, 16)]
                kvs = [jnp.take_along_axis(
                           kblk, jnp.full((16,), q, jnp.int32), axis=0)
                       for q in range(len(blk))]
                ts = [vx[_PAIRS_IJ[p][0]] * vx[_PAIRS_IJ[p][1]] for p in blk]
                ms = [t * kv for t, kv in zip(ts, kvs)]
                for p, m in zip(blk, ms):
                    plsc.addupdate(o_v.at[pl.ds(p * 16, 16)], m)

        for bl in range(CHUNK):
            for pc in range(0, PAIRS, 16):
                n = min(16, PAIRS - pc)
                idxv = tp_base + (pc * CHUNK + bl)
                if n == 16:
                    t_v[pl.ds(bl * PAIRS + pc, 16)] = plsc.load_gather(
                        o_v, [idxv])
                else:
                    mask = iota < n
                    v = plsc.load_gather(o_v, [idxv], mask=mask)
                    plsc.store_compressed(
                        t_v.at[pl.ds(bl * PAIRS + pc, 16)], v, mask=mask)

        cp = pltpu.make_async_copy(
            t_v.at[pl.ds(0, CHUNK * PAIRS)],
            o_hbm.at[pl.ds((base + c * CHUNK) * PAIRS, CHUNK * PAIRS)],
            so)
        cp.start()
        cp.wait()

        nxt = c + 2

        @pl.when(nxt < NCHUNK)
        def _():
            x_copy(nxt, b).start()

    def outer(cc, _):
        run_chunk(cc * 2, 0)
        run_chunk(cc * 2 + 1, 1)
        return 0

    lax.fori_loop(0, NCHUNK // 2, outer, 0)


_sc_call = pl.kernel(
    _sc_body,
    out_type=jax.ShapeDtypeStruct((B * PAIRS,), jnp.float32),
    mesh=plsc.VectorSubcoreMesh(core_axis_name="c", subcore_axis_name="s"),
    compiler_params=pltpu.CompilerParams(needs_layout_passes=False),
    scratch_types=[
        pltpu.VMEM((CHUNK * ROW,), jnp.float32),
        pltpu.VMEM((CHUNK * ROW,), jnp.float32),
        pltpu.VMEM((D * PAIRS + 16,), jnp.float32),
        pltpu.VMEM((PAIRS * CHUNK,), jnp.float32),
        pltpu.VMEM((CHUNK * PAIRS + 16,), jnp.float32),
        pltpu.SemaphoreType.DMA,
        pltpu.SemaphoreType.DMA,
        pltpu.SemaphoreType.DMA,
    ],
)


def kernel(inputs, kernel, training=False):
    b = inputs.shape[0]
    assert b == B and inputs.shape[1] == FIELD and inputs.shape[2] == D
    k_t = jnp.concatenate(
        [kernel.T.reshape(-1), jnp.zeros((16,), jnp.float32)])
    out = _sc_call(inputs.reshape(-1), k_t)
    return out.reshape(b, PAIRS)
